# baseline (speedup 1.0000x reference)
import jax
import jax.numpy as jnp
from jax import lax
from jax.experimental import pallas as pl
from jax.experimental.pallas import tpu as pltpu
from jax.experimental.pallas import tpu_sc as plsc

_GAMMA = 12.0
_HIDDEN = 128
_BATCH = 16384
_NC = 2
_NS = 16
_NW = _NC * _NS
_ROWS_PER_W = _BATCH // _NW
_CHUNK = 32
_NCHUNK = _ROWS_PER_W // _CHUNK
_ROW_UNROLL = 4


def _sc_body(head_hbm, rel_hbm, tail_hbm, ent_hbm, reltab_hbm, out_hbm,
             idx_h, idx_r, idx_t, hr3, t2, out_v, semh, semr, semt, semi):
    wid = lax.axis_index("s") * _NC + lax.axis_index("c")
    wbase = pl.multiple_of(wid * _ROWS_PER_W, _ROWS_PER_W)
    lane = lax.iota(jnp.int32, 16)

    icp0 = pltpu.async_copy(head_hbm.at[pl.ds(wbase, _ROWS_PER_W)], idx_h,
                            semi.at[0])
    icp1 = pltpu.async_copy(rel_hbm.at[pl.ds(wbase, _ROWS_PER_W)], idx_r,
                            semi.at[1])
    icp2 = pltpu.async_copy(tail_hbm.at[pl.ds(wbase, _ROWS_PER_W)], idx_t,
                            semi.at[2])
    icp0.wait()
    icp1.wait()
    icp2.wait()

    def h_desc(c):
        p = lax.rem(c, 3)
        sl = pl.ds(pl.multiple_of(c * _CHUNK, _CHUNK), _CHUNK)
        return pltpu.make_async_copy(ent_hbm.at[idx_h.at[sl]], hr3.at[p],
                                     semh.at[p])

    def r_desc(c):
        p = lax.rem(c, 3)
        sl = pl.ds(pl.multiple_of(c * _CHUNK, _CHUNK), _CHUNK)
        return pltpu.make_async_copy(reltab_hbm.at[idx_r.at[sl]], hr3.at[p],
                                     semr.at[p])

    def t_desc(c):
        p = lax.rem(c, 2)
        sl = pl.ds(pl.multiple_of(c * _CHUNK, _CHUNK), _CHUNK)
        return pltpu.make_async_copy(ent_hbm.at[idx_t.at[sl]], t2.at[p],
                                     semt.at[p])

    def launch_r(c):
        p = lax.rem(c, 3)
        sl = pl.ds(pl.multiple_of(c * _CHUNK, _CHUNK), _CHUNK)
        pltpu.async_copy(reltab_hbm.at[idx_r.at[sl]], hr3.at[p], semr.at[p],
                         add=True)

    for c in range(min(3, _NCHUNK)):
        h_desc(jnp.int32(c)).start()
    for c in range(min(2, _NCHUNK)):
        t_desc(jnp.int32(c)).start()
    h_desc(jnp.int32(0)).wait()
    launch_r(jnp.int32(0))

    def chunk_body(c, _):
        @pl.when(c + 1 < _NCHUNK)
        def _():
            h_desc(c + 1).wait()
            launch_r(c + 1)

        r_desc(c).wait()
        t_desc(c).wait()
        cp3 = lax.rem(c, 3)
        cp2 = lax.rem(c, 2)

        def group_body(g, _):
            def row_body(q, v):
                for u in range(_ROW_UNROLL):
                    rr = q * _ROW_UNROLL + u
                    row = g * 16 + rr
                    acc = jnp.zeros((16,), jnp.float32)
                    for k in range(_HIDDEN // 16):
                        sl = pl.ds(k * 16, 16)
                        acc = acc + jnp.abs(hr3[cp3, row, sl] -
                                            t2[cp2, row, sl])
                    s = _GAMMA - jnp.sum(acc)
                    v = jnp.where(lane == rr, s, v)
                return v

            v = lax.fori_loop(0, 16 // _ROW_UNROLL, row_body,
                              jnp.zeros((16,), jnp.float32))
            off = pl.multiple_of(c * _CHUNK + g * 16, 16)
            out_v[pl.ds(off, 16)] = v
            return 0

        lax.fori_loop(0, _CHUNK // 16, group_body, 0)

        @pl.when(c + 3 < _NCHUNK)
        def _():
            h_desc(c + 3).start()

        @pl.when(c + 2 < _NCHUNK)
        def _():
            t_desc(c + 2).start()

        return 0

    lax.fori_loop(0, _NCHUNK, chunk_body, 0)
    pltpu.sync_copy(out_v, out_hbm.at[pl.ds(wbase, _ROWS_PER_W)])


@jax.jit
def _run(head_idx, rel_idx, tail_idx, entity_embedding, relation_embedding):
    mesh = plsc.VectorSubcoreMesh(core_axis_name="c", subcore_axis_name="s")
    f = pl.kernel(
        _sc_body,
        out_type=jax.ShapeDtypeStruct((_BATCH,), jnp.float32),
        mesh=mesh,
        compiler_params=pltpu.CompilerParams(needs_layout_passes=False),
        scratch_types=[
            pltpu.VMEM((_ROWS_PER_W,), jnp.int32),
            pltpu.VMEM((_ROWS_PER_W,), jnp.int32),
            pltpu.VMEM((_ROWS_PER_W,), jnp.int32),
            pltpu.VMEM((3, _CHUNK, _HIDDEN), jnp.float32),
            pltpu.VMEM((2, _CHUNK, _HIDDEN), jnp.float32),
            pltpu.VMEM((_ROWS_PER_W,), jnp.float32),
            pltpu.SemaphoreType.DMA((3,)),
            pltpu.SemaphoreType.DMA((3,)),
            pltpu.SemaphoreType.DMA((2,)),
            pltpu.SemaphoreType.DMA((3,)),
        ],
    )
    return f(head_idx, rel_idx, tail_idx, entity_embedding,
             relation_embedding)


def kernel(sample, entity_embedding, relation_embedding):
    out = _run(sample[:, 0], sample[:, 1], sample[:, 2], entity_embedding,
               relation_embedding)
    return out[:, None]


# speedup vs baseline: 1.0385x; 1.0385x over previous
import jax
import jax.numpy as jnp
from jax import lax
from jax.experimental import pallas as pl
from jax.experimental.pallas import tpu as pltpu
from jax.experimental.pallas import tpu_sc as plsc

_GAMMA = 12.0
_HIDDEN = 128
_BATCH = 16384
_NC = 2
_NS = 16
_NW = _NC * _NS
_ROWS_PER_W = _BATCH // _NW
_CHUNK = 64
_NCHUNK = _ROWS_PER_W // _CHUNK
_ROW_UNROLL = 8


def _sc_body(head_hbm, rel_hbm, tail_hbm, ent_hbm, reltab_hbm, out_hbm,
             idx_h, idx_r, idx_t, hr3, t2, out_v, semh, semr, semt, semi):
    wid = lax.axis_index("s") * _NC + lax.axis_index("c")
    wbase = pl.multiple_of(wid * _ROWS_PER_W, _ROWS_PER_W)
    lane = lax.iota(jnp.int32, 16)

    icp0 = pltpu.async_copy(head_hbm.at[pl.ds(wbase, _ROWS_PER_W)], idx_h,
                            semi.at[0])
    icp1 = pltpu.async_copy(rel_hbm.at[pl.ds(wbase, _ROWS_PER_W)], idx_r,
                            semi.at[1])
    icp2 = pltpu.async_copy(tail_hbm.at[pl.ds(wbase, _ROWS_PER_W)], idx_t,
                            semi.at[2])
    icp0.wait()
    icp1.wait()
    icp2.wait()

    def h_desc(c):
        p = lax.rem(c, 3)
        sl = pl.ds(pl.multiple_of(c * _CHUNK, _CHUNK), _CHUNK)
        return pltpu.make_async_copy(ent_hbm.at[idx_h.at[sl]], hr3.at[p],
                                     semh.at[p])

    def r_desc(c):
        p = lax.rem(c, 3)
        sl = pl.ds(pl.multiple_of(c * _CHUNK, _CHUNK), _CHUNK)
        return pltpu.make_async_copy(reltab_hbm.at[idx_r.at[sl]], hr3.at[p],
                                     semr.at[p])

    def t_desc(c):
        p = lax.rem(c, 2)
        sl = pl.ds(pl.multiple_of(c * _CHUNK, _CHUNK), _CHUNK)
        return pltpu.make_async_copy(ent_hbm.at[idx_t.at[sl]], t2.at[p],
                                     semt.at[p])

    def launch_r(c):
        p = lax.rem(c, 3)
        sl = pl.ds(pl.multiple_of(c * _CHUNK, _CHUNK), _CHUNK)
        pltpu.async_copy(reltab_hbm.at[idx_r.at[sl]], hr3.at[p], semr.at[p],
                         add=True)

    for c in range(min(3, _NCHUNK)):
        h_desc(jnp.int32(c)).start()
    for c in range(min(2, _NCHUNK)):
        t_desc(jnp.int32(c)).start()
    h_desc(jnp.int32(0)).wait()
    launch_r(jnp.int32(0))

    def chunk_body(c, _):
        @pl.when(c + 1 < _NCHUNK)
        def _():
            h_desc(c + 1).wait()
            launch_r(c + 1)

        r_desc(c).wait()
        t_desc(c).wait()
        cp3 = lax.rem(c, 3)
        cp2 = lax.rem(c, 2)

        def group_body(g, _):
            def row_body(q, v):
                for u in range(_ROW_UNROLL):
                    rr = q * _ROW_UNROLL + u
                    row = g * 16 + rr
                    acc = jnp.zeros((16,), jnp.float32)
                    for k in range(_HIDDEN // 16):
                        sl = pl.ds(k * 16, 16)
                        acc = acc + jnp.abs(hr3[cp3, row, sl] -
                                            t2[cp2, row, sl])
                    s = _GAMMA - jnp.sum(acc)
                    v = jnp.where(lane == rr, s, v)
                return v

            v = lax.fori_loop(0, 16 // _ROW_UNROLL, row_body,
                              jnp.zeros((16,), jnp.float32))
            off = pl.multiple_of(c * _CHUNK + g * 16, 16)
            out_v[pl.ds(off, 16)] = v
            return 0

        lax.fori_loop(0, _CHUNK // 16, group_body, 0)

        @pl.when(c + 3 < _NCHUNK)
        def _():
            h_desc(c + 3).start()

        @pl.when(c + 2 < _NCHUNK)
        def _():
            t_desc(c + 2).start()

        return 0

    lax.fori_loop(0, _NCHUNK, chunk_body, 0)
    pltpu.sync_copy(out_v, out_hbm.at[pl.ds(wbase, _ROWS_PER_W)])


@jax.jit
def _run(head_idx, rel_idx, tail_idx, entity_embedding, relation_embedding):
    mesh = plsc.VectorSubcoreMesh(core_axis_name="c", subcore_axis_name="s")
    f = pl.kernel(
        _sc_body,
        out_type=jax.ShapeDtypeStruct((_BATCH,), jnp.float32),
        mesh=mesh,
        compiler_params=pltpu.CompilerParams(needs_layout_passes=False),
        scratch_types=[
            pltpu.VMEM((_ROWS_PER_W,), jnp.int32),
            pltpu.VMEM((_ROWS_PER_W,), jnp.int32),
            pltpu.VMEM((_ROWS_PER_W,), jnp.int32),
            pltpu.VMEM((3, _CHUNK, _HIDDEN), jnp.float32),
            pltpu.VMEM((2, _CHUNK, _HIDDEN), jnp.float32),
            pltpu.VMEM((_ROWS_PER_W,), jnp.float32),
            pltpu.SemaphoreType.DMA((3,)),
            pltpu.SemaphoreType.DMA((3,)),
            pltpu.SemaphoreType.DMA((2,)),
            pltpu.SemaphoreType.DMA((3,)),
        ],
    )
    return f(head_idx, rel_idx, tail_idx, entity_embedding,
             relation_embedding)


def kernel(sample, entity_embedding, relation_embedding):
    out = _run(sample[:, 0], sample[:, 1], sample[:, 2], entity_embedding,
               relation_embedding)
    return out[:, None]


# speedup vs baseline: 1.0479x; 1.0091x over previous
import jax
import jax.numpy as jnp
from jax import lax
from jax.experimental import pallas as pl
from jax.experimental.pallas import tpu as pltpu
from jax.experimental.pallas import tpu_sc as plsc

_GAMMA = 12.0
_HIDDEN = 128
_BATCH = 16384
_NC = 2
_NS = 16
_NW = _NC * _NS
_ROWS_PER_W = _BATCH // _NW
_CHUNK = 64
_NCHUNK = _ROWS_PER_W // _CHUNK
_ROW_UNROLL = 4


def _sc_body(head_hbm, rel_hbm, tail_hbm, ent_hbm, reltab_hbm, out_hbm,
             idx_h, idx_r, idx_t, hr3, t2, out_v, semh, semr, semt, semi):
    wid = lax.axis_index("s") * _NC + lax.axis_index("c")
    wbase = pl.multiple_of(wid * _ROWS_PER_W, _ROWS_PER_W)
    lane = lax.iota(jnp.int32, 16)

    icp0 = pltpu.async_copy(head_hbm.at[pl.ds(wbase, _ROWS_PER_W)], idx_h,
                            semi.at[0])
    icp1 = pltpu.async_copy(rel_hbm.at[pl.ds(wbase, _ROWS_PER_W)], idx_r,
                            semi.at[1])
    icp2 = pltpu.async_copy(tail_hbm.at[pl.ds(wbase, _ROWS_PER_W)], idx_t,
                            semi.at[2])

    def h_desc(c):
        p = lax.rem(c, 3)
        sl = pl.ds(pl.multiple_of(c * _CHUNK, _CHUNK), _CHUNK)
        return pltpu.make_async_copy(ent_hbm.at[idx_h.at[sl]], hr3.at[p],
                                     semh.at[p])

    def r_desc(c):
        p = lax.rem(c, 3)
        sl = pl.ds(pl.multiple_of(c * _CHUNK, _CHUNK), _CHUNK)
        return pltpu.make_async_copy(reltab_hbm.at[idx_r.at[sl]], hr3.at[p],
                                     semr.at[p])

    def t_desc(c):
        p = lax.rem(c, 2)
        sl = pl.ds(pl.multiple_of(c * _CHUNK, _CHUNK), _CHUNK)
        return pltpu.make_async_copy(ent_hbm.at[idx_t.at[sl]], t2.at[p],
                                     semt.at[p])

    def launch_r(c):
        p = lax.rem(c, 3)
        sl = pl.ds(pl.multiple_of(c * _CHUNK, _CHUNK), _CHUNK)
        pltpu.async_copy(reltab_hbm.at[idx_r.at[sl]], hr3.at[p], semr.at[p],
                         add=True)

    icp0.wait()
    for c in range(min(3, _NCHUNK)):
        h_desc(jnp.int32(c)).start()
    icp2.wait()
    for c in range(min(2, _NCHUNK)):
        t_desc(jnp.int32(c)).start()
    icp1.wait()
    h_desc(jnp.int32(0)).wait()
    launch_r(jnp.int32(0))

    def chunk_body(c, _):
        @pl.when(c + 1 < _NCHUNK)
        def _():
            h_desc(c + 1).wait()
            launch_r(c + 1)

        r_desc(c).wait()
        t_desc(c).wait()
        cp3 = lax.rem(c, 3)
        cp2 = lax.rem(c, 2)

        def group_body(g, _):
            def row_body(q, v):
                for u in range(_ROW_UNROLL):
                    rr = q * _ROW_UNROLL + u
                    row = g * 16 + rr
                    acc = jnp.zeros((16,), jnp.float32)
                    for k in range(_HIDDEN // 16):
                        sl = pl.ds(k * 16, 16)
                        acc = acc + jnp.abs(hr3[cp3, row, sl] -
                                            t2[cp2, row, sl])
                    s = _GAMMA - jnp.sum(acc)
                    v = jnp.where(lane == rr, s, v)
                return v

            v = lax.fori_loop(0, 16 // _ROW_UNROLL, row_body,
                              jnp.zeros((16,), jnp.float32))
            off = pl.multiple_of(c * _CHUNK + g * 16, 16)
            out_v[pl.ds(off, 16)] = v
            return 0

        lax.fori_loop(0, _CHUNK // 16, group_body, 0)

        @pl.when(c + 3 < _NCHUNK)
        def _():
            h_desc(c + 3).start()

        @pl.when(c + 2 < _NCHUNK)
        def _():
            t_desc(c + 2).start()

        return 0

    lax.fori_loop(0, _NCHUNK, chunk_body, 0)
    pltpu.sync_copy(out_v, out_hbm.at[pl.ds(wbase, _ROWS_PER_W)])


@jax.jit
def _run(head_idx, rel_idx, tail_idx, entity_embedding, relation_embedding):
    mesh = plsc.VectorSubcoreMesh(core_axis_name="c", subcore_axis_name="s")
    f = pl.kernel(
        _sc_body,
        out_type=jax.ShapeDtypeStruct((_BATCH,), jnp.float32),
        mesh=mesh,
        compiler_params=pltpu.CompilerParams(needs_layout_passes=False),
        scratch_types=[
            pltpu.VMEM((_ROWS_PER_W,), jnp.int32),
            pltpu.VMEM((_ROWS_PER_W,), jnp.int32),
            pltpu.VMEM((_ROWS_PER_W,), jnp.int32),
            pltpu.VMEM((3, _CHUNK, _HIDDEN), jnp.float32),
            pltpu.VMEM((2, _CHUNK, _HIDDEN), jnp.float32),
            pltpu.VMEM((_ROWS_PER_W,), jnp.float32),
            pltpu.SemaphoreType.DMA((3,)),
            pltpu.SemaphoreType.DMA((3,)),
            pltpu.SemaphoreType.DMA((2,)),
            pltpu.SemaphoreType.DMA((3,)),
        ],
    )
    return f(head_idx, rel_idx, tail_idx, entity_embedding,
             relation_embedding)


def kernel(sample, entity_embedding, relation_embedding):
    out = _run(sample[:, 0], sample[:, 1], sample[:, 2], entity_embedding,
               relation_embedding)
    return out[:, None]
